# Initial kernel scaffold; baseline (speedup 1.0000x reference)
import jax
import jax.numpy as jnp
from jax.experimental import pallas as pl


def kernel(x, edge_index, W_in, b_in, W0, as0, ad0, bg0, g0, bb0, W1, as1, ad1, bg1, g1, bb1, W2, as2, ad2, bg2, g2, bb2, W_out, b_out):
    raise NotImplementedError("write your pallas kernel here")


# speedup vs baseline: 86.6464x; 86.6464x over previous
import functools

import jax
import jax.numpy as jnp
from jax import lax
from jax.experimental import pallas as pl
from jax.experimental.pallas import tpu as pltpu
from jax.experimental.pallas import tpu_sc as plsc

N = 10000
E = 320000
D_IN = 128
H = 8
C = 8
HID = 64

NC = 2
NS = 16
NW = NC * NS
LANES = 16

CH = 128
ETOT = E + N
CPW = -(-ETOT // (NW * CH))
EPW = CPW * CH
EPAD = EPW * NW
DROW = 80
NACC = 10240
ROWS_PER_TILE = NACC // NS
RCHUNK = 128

_EPS_DIV = 1e-16
_EPS_LN = 1e-5


def _expand_mat():
    col = lax.broadcasted_iota(jnp.int32, (H, DROW), 1)
    row = lax.broadcasted_iota(jnp.int32, (H, DROW), 0)
    head = jnp.where(col < HID, col // C, col - HID)
    return ((head == row) & (col < HID + H)).astype(jnp.float32)


def _attn_tail(h, W, As_mat, Ad_mat, xe_ref, ase_ref, ade_ref, m80_ref):
    xh = jnp.dot(h, W, preferred_element_type=jnp.float32)
    asrc = jnp.dot(xh, As_mat, preferred_element_type=jnp.float32)
    adst = jnp.dot(xh, Ad_mat, preferred_element_type=jnp.float32)
    xe_ref[...] = jnp.concatenate(
        [xh, jnp.ones((N, H), jnp.float32), jnp.zeros((N, H), jnp.float32)],
        axis=1)
    expm = _expand_mat()
    ase_ref[...] = jnp.dot(asrc, expm, preferred_element_type=jnp.float32)
    ade_ref[...] = jnp.dot(adst, expm, preferred_element_type=jnp.float32)
    m8 = (jnp.max(asrc, axis=0, keepdims=True)
          + jnp.max(adst, axis=0, keepdims=True))
    m80_ref[...] = jnp.dot(m8, expm, preferred_element_type=jnp.float32)


def _tc_pre_body(x_ref, Win_ref, bin_ref, W_ref, Asm_ref, Adm_ref,
                 h_ref, xe_ref, ase_ref, ade_ref, m80_ref):
    h = jnp.maximum(
        jnp.dot(x_ref[...], Win_ref[...], preferred_element_type=jnp.float32)
        + bin_ref[...], 0.0)
    h_ref[...] = h
    _attn_tail(h, W_ref[...], Asm_ref[...], Adm_ref[...],
               xe_ref, ase_ref, ade_ref, m80_ref)


def _combine(p0, p1, hprev, bg, g, bb):
    acc = p0[:N] + p1[:N]
    out_un = acc[:, :HID]
    s = acc[:, HID:HID + H]
    sinv = 1.0 / (s + _EPS_DIV)
    col = lax.broadcasted_iota(jnp.int32, (H, HID), 1) // C
    row = lax.broadcasted_iota(jnp.int32, (H, HID), 0)
    expm = (col == row).astype(jnp.float32)
    hn = out_un * jnp.dot(sinv, expm, preferred_element_type=jnp.float32) + bg
    mu = jnp.mean(hn, axis=-1, keepdims=True)
    var = jnp.mean((hn - mu) ** 2, axis=-1, keepdims=True)
    hn = (hn - mu) / jnp.sqrt(var + _EPS_LN) * g + bb
    return hprev + jnp.maximum(hn, 0.0)


def _tc_mid_body(p0_ref, p1_ref, hprev_ref, bg_ref, g_ref, bb_ref,
                 W_ref, Asm_ref, Adm_ref,
                 h_ref, xe_ref, ase_ref, ade_ref, m80_ref):
    h = _combine(p0_ref[...], p1_ref[...], hprev_ref[...],
                 bg_ref[...], g_ref[...], bb_ref[...])
    h_ref[...] = h
    _attn_tail(h, W_ref[...], Asm_ref[...], Adm_ref[...],
               xe_ref, ase_ref, ade_ref, m80_ref)


def _tc_final_body(p0_ref, p1_ref, hprev_ref, bg_ref, g_ref, bb_ref,
                   Wout_ref, bout_ref, y_ref):
    h = _combine(p0_ref[...], p1_ref[...], hprev_ref[...],
                 bg_ref[...], g_ref[...], bb_ref[...])
    y_ref[...] = (jnp.dot(h, Wout_ref[...], preferred_element_type=jnp.float32)
                  + bout_ref[...])


_NODE_OUTS = [
    jax.ShapeDtypeStruct((N, HID), jnp.float32),
    jax.ShapeDtypeStruct((N, DROW), jnp.float32),
    jax.ShapeDtypeStruct((N, DROW), jnp.float32),
    jax.ShapeDtypeStruct((N, DROW), jnp.float32),
    jax.ShapeDtypeStruct((1, DROW), jnp.float32),
]

_tc_pre = pl.pallas_call(_tc_pre_body, out_shape=_NODE_OUTS)
_tc_mid = pl.pallas_call(_tc_mid_body, out_shape=_NODE_OUTS)
_tc_final = pl.pallas_call(
    _tc_final_body, out_shape=jax.ShapeDtypeStruct((N, 1), jnp.float32))


def _sc_edge_body(src_hbm, dst_hbm, ase_hbm, ade_hbm, big_hbm, m_hbm,
                  out_hbm,
                  srcv, dstv, r1, r2, xrows, wbuf, mv, accum,
                  sem1, sem2, sem3):
    c = lax.axis_index("c")
    s = lax.axis_index("s")
    wid = s * NC + c

    def _zero_row(r, carry):
        for j in range(DROW // LANES):
            wbuf[r, pl.ds(LANES * j, LANES)] = jnp.zeros((LANES,), jnp.float32)
        return carry
    lax.fori_loop(0, CH, _zero_row, 0)
    row0 = s * ROWS_PER_TILE
    for k in range(ROWS_PER_TILE // RCHUNK):
        pltpu.sync_copy(wbuf.at[pl.ds(0, RCHUNK)],
                        accum.at[pl.ds(row0 + k * RCHUNK, RCHUNK)])
    plsc.subcore_barrier()

    pltpu.sync_copy(m_hbm, mv)
    mvs = [mv[pl.ds(LANES * j, LANES)] for j in range(DROW // LANES)]

    base0 = wid * EPW

    def _chunk(ch, carry):
        base = base0 + ch * CH
        pltpu.sync_copy(src_hbm.at[pl.ds(base, CH)], srcv)
        pltpu.sync_copy(dst_hbm.at[pl.ds(base, CH)], dstv)
        cp1 = pltpu.async_copy(ase_hbm.at[srcv], r1, sem1)
        cp2 = pltpu.async_copy(ade_hbm.at[dstv], r2, sem2)
        cp3 = pltpu.async_copy(big_hbm.at[srcv], xrows, sem3)
        cp1.wait()
        cp2.wait()
        cp3.wait()

        def _edge(e, carry2):
            for j in range(DROW // LANES):
                sl = pl.ds(LANES * j, LANES)
                t = r1[e, sl] + r2[e, sl]
                t = jnp.maximum(t, 0.0) + 0.2 * jnp.minimum(t, 0.0)
                ev = jnp.exp(t - mvs[j])
                wbuf[e, sl] = xrows[e, sl] * ev
            return carry2
        lax.fori_loop(0, CH, _edge, 0)

        pltpu.sync_copy(wbuf, accum.at[dstv], add=True)
        return carry
    lax.fori_loop(0, CPW, _chunk, 0)

    plsc.subcore_barrier()

    for k in range(ROWS_PER_TILE // RCHUNK):
        r0 = row0 + k * RCHUNK
        pltpu.sync_copy(accum.at[pl.ds(r0, RCHUNK)],
                        out_hbm.at[c, pl.ds(r0, RCHUNK)])


@functools.cache
def _make_sc_edge():
  return pl.kernel(
    _sc_edge_body,
    compiler_params=pltpu.CompilerParams(use_tc_tiling_on_sc=False),
    out_type=jax.ShapeDtypeStruct((NC, NACC, DROW), jnp.float32),
    mesh=plsc.VectorSubcoreMesh(core_axis_name="c", subcore_axis_name="s",
                                num_cores=NC, num_subcores=NS),
    scratch_types=[
        pltpu.VMEM((CH,), jnp.int32),
        pltpu.VMEM((CH,), jnp.int32),
        pltpu.VMEM((CH, DROW), jnp.float32),
        pltpu.VMEM((CH, DROW), jnp.float32),
        pltpu.VMEM((CH, DROW), jnp.float32),
        pltpu.VMEM((CH, DROW), jnp.float32),
        pltpu.VMEM((DROW,), jnp.float32),
        pltpu.VMEM_SHARED((NACC, DROW), jnp.float32),
        pltpu.SemaphoreType.DMA,
        pltpu.SemaphoreType.DMA,
        pltpu.SemaphoreType.DMA,
    ],
  )


def _expand_a(a):
    k = jnp.arange(HID)
    return jnp.zeros((HID, H), jnp.float32).at[k, k // C].set(a.reshape(HID))


def kernel(x, edge_index, W_in, b_in, W0, as0, ad0, bg0, g0, bb0,
           W1, as1, ad1, bg1, g1, bb1, W2, as2, ad2, bg2, g2, bb2,
           W_out, b_out):
    idt = edge_index.dtype
    loop = jnp.arange(N, dtype=idt)
    npad = EPAD - ETOT
    src = jnp.concatenate(
        [edge_index[0], loop, jnp.full((npad,), N, dtype=idt)])
    dst = jnp.concatenate(
        [edge_index[1], loop, jnp.zeros((npad,), dtype=idt)])
    src = src.astype(jnp.int32)
    dst = dst.astype(jnp.int32)

    layers = [(W0, as0, ad0, bg0, g0, bb0), (W1, as1, ad1, bg1, g1, bb1),
              (W2, as2, ad2, bg2, g2, bb2)]

    b_in2 = b_in.reshape(1, HID)
    W, a_s, a_d = layers[0][0], layers[0][1], layers[0][2]
    h, xe, ase, ade, m80 = _tc_pre(x, W_in, b_in2, W,
                                   _expand_a(a_s), _expand_a(a_d))

    for l in range(3):
        big = jnp.pad(xe, ((0, 1), (0, 0)))
        ase_p = jnp.pad(ase, ((0, 1), (0, 0)))
        parts = _make_sc_edge()(src, dst, ase_p, ade, big, m80.reshape(DROW))
        p0, p1 = parts[0], parts[1]
        bg, g, bb = layers[l][3], layers[l][4], layers[l][5]
        bg2, g2_, bb2 = (bg.reshape(1, HID), g.reshape(1, HID),
                         bb.reshape(1, HID))
        if l < 2:
            Wn, asn, adn = layers[l + 1][0], layers[l + 1][1], layers[l + 1][2]
            h, xe, ase, ade, m80 = _tc_mid(p0, p1, h, bg2, g2_, bb2, Wn,
                                           _expand_a(asn), _expand_a(adn))
        else:
            y = _tc_final(p0, p1, h, bg2, g2_, bb2, W_out,
                          b_out.reshape(1, 1))
    return y
